# baseline (speedup 1.0000x reference)
import math

import jax
import jax.numpy as jnp
from jax.experimental import pallas as pl
from jax.experimental.pallas import tpu as pltpu

_NINP = 1024
_NB = 8
_BH = 256
_TOPK = 2
_DK = 64
_DV = 256
_DKC = 32


def _precompute_body(x_ref, wk_ref, wv_ref, kx_ref, v0_ref):
    x = x_ref[...]
    kx = jnp.dot(x, wk_ref[...], preferred_element_type=jnp.float32)
    kx_ref[...] = kx.astype(jnp.bfloat16)
    v0 = jnp.dot(x, wv_ref[...], preferred_element_type=jnp.float32)
    v0_ref[...] = v0.astype(jnp.bfloat16)


def _step_body(*refs):
    (kx_ref, v0_ref, h0_ref) = refs[0:3]
    wq_refs = refs[3:3 + _NB]
    wh_refs = refs[3 + _NB:3 + 2 * _NB]
    wx_refs = refs[3 + 2 * _NB:3 + 3 * _NB]
    wqc_refs = refs[3 + 3 * _NB:3 + 4 * _NB]
    wkc_refs = refs[3 + 4 * _NB:3 + 5 * _NB]
    wvc_refs = refs[3 + 5 * _NB:3 + 6 * _NB]
    bq_ref, bg_ref = refs[3 + 6 * _NB:5 + 6 * _NB]
    out_ref = refs[5 + 6 * _NB]
    h_scr = refs[6 + 6 * _NB]
    hbf_scr = refs[7 + 6 * _NB]
    hnbf_scr = refs[8 + 6 * _NB]
    xin_scr = refs[9 + 6 * _NB]

    t = pl.program_id(0)

    @pl.when(t == 0)
    def _init():
        h_scr[...] = h0_ref[...]
        hbf_scr[...] = h0_ref[...].astype(jnp.bfloat16)

    kx = kx_ref[0].astype(jnp.float32)
    v0 = v0_ref[0].astype(jnp.float32)

    h_list = [h_scr[n] for n in range(_NB)]
    s_cols = []
    gh_list = []
    for n in range(_NB):
        q = jnp.dot(hbf_scr[n], wq_refs[n][...],
                    preferred_element_type=jnp.float32) + bq_ref[n]
        qb = q.astype(jnp.bfloat16).astype(jnp.float32)
        s_cols.append(jnp.sum(qb * kx, axis=1, keepdims=True))
        gh_list.append(jnp.dot(hbf_scr[n], wh_refs[n][...],
                               preferred_element_type=jnp.float32))
    s = jnp.concatenate(s_cols, axis=1) / 8.0

    m = jnp.maximum(s, 0.0)
    e = jnp.exp(s - m)
    att0 = e / (e + jnp.exp(-m))
    attb = att0.astype(jnp.bfloat16).astype(jnp.float32)

    hn_list = []
    for n in range(_NB):
        xin_scr[n] = (attb[:, n:n + 1] * v0).astype(jnp.bfloat16)
    for n in range(_NB):
        gx = jnp.dot(xin_scr[n], wx_refs[n][...],
                     preferred_element_type=jnp.float32) + bg_ref[n]
        gh = gh_list[n]
        r = jax.nn.sigmoid(gx[:, :_BH] + gh[:, :_BH])
        z = jax.nn.sigmoid(gx[:, _BH:2 * _BH] + gh[:, _BH:2 * _BH])
        g = jnp.tanh(gx[:, 2 * _BH:] + r * gh[:, 2 * _BH:])
        hn = (1.0 - z) * g + z * h_list[n]
        hn_list.append(hn)
        hnbf_scr[n] = hn.astype(jnp.bfloat16)

    qc_list, kc_list, vc_list = [], [], []
    for n in range(_NB):
        qc = jnp.dot(hnbf_scr[n], wqc_refs[n][...],
                     preferred_element_type=jnp.float32)
        kc = jnp.dot(hnbf_scr[n], wkc_refs[n][...],
                     preferred_element_type=jnp.float32)
        vc = jnp.dot(hnbf_scr[n], wvc_refs[n][...],
                     preferred_element_type=jnp.float32)
        qc_list.append(qc.astype(jnp.bfloat16).astype(jnp.float32))
        kc_list.append(kc.astype(jnp.bfloat16).astype(jnp.float32))
        vc_list.append(vc)
    qcs = jnp.stack(qc_list)
    kcs = jnp.stack(kc_list)
    vcs = jnp.stack(vc_list)
    logits = (jnp.sum(qcs[:, None] * kcs[None, :], axis=-1)
              / math.sqrt(_DKC))
    lmax = jnp.max(logits, axis=1, keepdims=True)
    le = jnp.exp(logits - lmax)
    ac = le / jnp.sum(le, axis=1, keepdims=True)
    acb = ac.astype(jnp.bfloat16).astype(jnp.float32)
    vcb = vcs.astype(jnp.bfloat16).astype(jnp.float32)

    r1 = att0[:, None, :]
    r2 = att0[:, :, None]
    n_idx = jax.lax.broadcasted_iota(jnp.int32, (1, _NB, _NB), 1)
    m_idx = jax.lax.broadcasted_iota(jnp.int32, (1, _NB, _NB), 2)
    beats = (r1 > r2) | ((r1 == r2) & (m_idx < n_idx))
    rank = jnp.sum(beats.astype(jnp.int32), axis=2)
    maskf = (rank < _TOPK).astype(jnp.float32)

    for n in range(_NB):
        comm = jnp.sum(acb[n][:, :, None] * vcb, axis=0)
        hn2 = hn_list[n] + comm
        mk = maskf[:, n:n + 1]
        hout = mk * hn2 + (1.0 - mk) * h_list[n]
        h_scr[n] = hout
        hbf_scr[n] = hout.astype(jnp.bfloat16)
        out_ref[0, :, n * _BH:(n + 1) * _BH] = hout


def _full(shape):
    nd = len(shape)
    return pl.BlockSpec(shape, lambda t, _nd=nd: (0,) * _nd)


def kernel(input, hidden, seq_len, Wq_in, bq_in, Wk_in, Wv_in, Wx, Wh, bg,
           Wq_c, Wk_c, Wv_c):
    seq, batch, ninp = input.shape
    rows = seq * batch
    tile = 256
    grid_pre = rows // tile

    xf = input.reshape(rows, ninp).astype(jnp.bfloat16)
    kx_flat, v0_flat = pl.pallas_call(
        _precompute_body,
        grid=(grid_pre,),
        in_specs=[
            pl.BlockSpec((tile, ninp), lambda i: (i, 0)),
            pl.BlockSpec((ninp, _DK), lambda i: (0, 0)),
            pl.BlockSpec((ninp, _DV), lambda i: (0, 0)),
        ],
        out_specs=[
            pl.BlockSpec((tile, _DK), lambda i: (i, 0)),
            pl.BlockSpec((tile, _DV), lambda i: (i, 0)),
        ],
        out_shape=[
            jax.ShapeDtypeStruct((rows, _DK), jnp.bfloat16),
            jax.ShapeDtypeStruct((rows, _DV), jnp.bfloat16),
        ],
    )(xf, Wk_in.astype(jnp.bfloat16), Wv_in.astype(jnp.bfloat16))

    kx_v = kx_flat.reshape(seq, batch, _DK)
    v0_v = v0_flat.reshape(seq, batch, _DV)
    h0 = hidden.reshape(batch, _NB, _BH).transpose(1, 0, 2)

    wq_l = [Wq_in[n].astype(jnp.bfloat16) for n in range(_NB)]
    wh_l = [Wh[n].astype(jnp.bfloat16) for n in range(_NB)]
    wx_l = [Wx[n].astype(jnp.bfloat16) for n in range(_NB)]
    wqc_l = [Wq_c[n].astype(jnp.bfloat16) for n in range(_NB)]
    wkc_l = [Wk_c[n].astype(jnp.bfloat16) for n in range(_NB)]
    wvc_l = [Wv_c[n].astype(jnp.bfloat16) for n in range(_NB)]

    in_specs = [
        pl.BlockSpec((1, batch, _DK), lambda t: (t, 0, 0)),
        pl.BlockSpec((1, batch, _DV), lambda t: (t, 0, 0)),
        _full((_NB, batch, _BH)),
    ]
    in_specs += [_full((_BH, _DK))] * _NB
    in_specs += [_full((_BH, 3 * _BH))] * _NB
    in_specs += [_full((_DV, 3 * _BH))] * _NB
    in_specs += [_full((_BH, _DKC))] * _NB
    in_specs += [_full((_BH, _DKC))] * _NB
    in_specs += [_full((_BH, _BH))] * _NB
    in_specs += [_full((_NB, _DK)), _full((_NB, 3 * _BH))]

    out = pl.pallas_call(
        _step_body,
        grid=(seq,),
        in_specs=in_specs,
        out_specs=pl.BlockSpec((1, batch, _NB * _BH), lambda t: (t, 0, 0)),
        out_shape=jax.ShapeDtypeStruct((seq, batch, _NB * _BH), jnp.float32),
        scratch_shapes=[pltpu.VMEM((_NB, batch, _BH), jnp.float32),
                        pltpu.VMEM((_NB, batch, _BH), jnp.bfloat16),
                        pltpu.VMEM((_NB, batch, _BH), jnp.bfloat16),
                        pltpu.VMEM((_NB, batch, _DV), jnp.bfloat16)],
        compiler_params=pltpu.CompilerParams(
            dimension_semantics=("arbitrary",),
        ),
    )(kx_v, v0_v, h0, *wq_l, *wh_l, *wx_l, *wqc_l, *wkc_l, *wvc_l, bq_in, bg)

    return out


# speedup vs baseline: 1.2634x; 1.2634x over previous
import math

import jax
import jax.numpy as jnp
from jax.experimental import pallas as pl
from jax.experimental.pallas import tpu as pltpu

_NINP = 1024
_NB = 8
_BH = 256
_TOPK = 2
_DK = 64
_DV = 256
_DKC = 32


def _precompute_body(x_ref, wk_ref, wv_ref, kx_ref, v0_ref):
    x = x_ref[...]
    kx = jnp.dot(x, wk_ref[...], preferred_element_type=jnp.float32)
    kx_ref[...] = kx.astype(jnp.bfloat16)
    v0 = jnp.dot(x, wv_ref[...], preferred_element_type=jnp.float32)
    v0_ref[...] = v0.astype(jnp.bfloat16)


def _step_body(*refs):
    (kx_ref, v0_ref, h0_ref) = refs[0:3]
    wq_refs = refs[3:3 + _NB]
    wh_refs = refs[3 + _NB:3 + 2 * _NB]
    wx_refs = refs[3 + 2 * _NB:3 + 3 * _NB]
    wqc_refs = refs[3 + 3 * _NB:3 + 4 * _NB]
    wkc_refs = refs[3 + 4 * _NB:3 + 5 * _NB]
    wvc_refs = refs[3 + 5 * _NB:3 + 6 * _NB]
    bq_ref, bg_ref = refs[3 + 6 * _NB:5 + 6 * _NB]
    out_ref = refs[5 + 6 * _NB]
    h_scr = refs[6 + 6 * _NB]

    t = pl.program_id(0)

    @pl.when(t == 0)
    def _init():
        h_scr[...] = h0_ref[...]

    kx = kx_ref[0].astype(jnp.float32)
    v0 = v0_ref[0].astype(jnp.float32)

    h_list = [h_scr[n] for n in range(_NB)]
    s_cols = []
    gh_list = []
    for n in range(_NB):
        q = jnp.dot(h_list[n], wq_refs[n][...],
                    preferred_element_type=jnp.float32) + bq_ref[n]
        qb = q.astype(jnp.bfloat16).astype(jnp.float32)
        s_cols.append(jnp.sum(qb * kx, axis=1, keepdims=True))
        gh_list.append(jnp.dot(h_list[n], wh_refs[n][...],
                               preferred_element_type=jnp.float32))
    s = jnp.concatenate(s_cols, axis=1) / 8.0

    m = jnp.maximum(s, 0.0)
    e = jnp.exp(s - m)
    att0 = e / (e + jnp.exp(-m))
    attb = att0.astype(jnp.bfloat16).astype(jnp.float32)

    hn_list = []
    for n in range(_NB):
        xin = attb[:, n:n + 1] * v0
        gx = jnp.dot(xin, wx_refs[n][...],
                     preferred_element_type=jnp.float32) + bg_ref[n]
        gh = gh_list[n]
        r = jax.nn.sigmoid(gx[:, :_BH] + gh[:, :_BH])
        z = jax.nn.sigmoid(gx[:, _BH:2 * _BH] + gh[:, _BH:2 * _BH])
        g = jnp.tanh(gx[:, 2 * _BH:] + r * gh[:, 2 * _BH:])
        hn_list.append((1.0 - z) * g + z * h_list[n])

    qc_list, kc_list, vc_list = [], [], []
    for n in range(_NB):
        qc = jnp.dot(hn_list[n], wqc_refs[n][...],
                     preferred_element_type=jnp.float32)
        kc = jnp.dot(hn_list[n], wkc_refs[n][...],
                     preferred_element_type=jnp.float32)
        vc = jnp.dot(hn_list[n], wvc_refs[n][...],
                     preferred_element_type=jnp.float32)
        qc_list.append(qc.astype(jnp.bfloat16).astype(jnp.float32))
        kc_list.append(kc.astype(jnp.bfloat16).astype(jnp.float32))
        vc_list.append(vc)
    qcs = jnp.stack(qc_list)
    kcs = jnp.stack(kc_list)
    vcs = jnp.stack(vc_list)
    logits = (jnp.sum(qcs[:, None] * kcs[None, :], axis=-1)
              / math.sqrt(_DKC))
    lmax = jnp.max(logits, axis=1, keepdims=True)
    le = jnp.exp(logits - lmax)
    ac = le / jnp.sum(le, axis=1, keepdims=True)
    acb = ac.astype(jnp.bfloat16).astype(jnp.float32)
    vcb = vcs.astype(jnp.bfloat16).astype(jnp.float32)

    r1 = att0[:, None, :]
    r2 = att0[:, :, None]
    n_idx = jax.lax.broadcasted_iota(jnp.int32, (1, _NB, _NB), 1)
    m_idx = jax.lax.broadcasted_iota(jnp.int32, (1, _NB, _NB), 2)
    beats = (r1 > r2) | ((r1 == r2) & (m_idx < n_idx))
    rank = jnp.sum(beats.astype(jnp.int32), axis=2)
    maskf = (rank < _TOPK).astype(jnp.float32)

    for n in range(_NB):
        comm = jnp.sum(acb[n][:, :, None] * vcb, axis=0)
        hn2 = hn_list[n] + comm
        mk = maskf[:, n:n + 1]
        hout = mk * hn2 + (1.0 - mk) * h_list[n]
        h_scr[n] = hout
        out_ref[0, :, n * _BH:(n + 1) * _BH] = hout


def _full(shape):
    nd = len(shape)
    return pl.BlockSpec(shape, lambda t, _nd=nd: (0,) * _nd)


def kernel(input, hidden, seq_len, Wq_in, bq_in, Wk_in, Wv_in, Wx, Wh, bg,
           Wq_c, Wk_c, Wv_c):
    seq, batch, ninp = input.shape
    rows = seq * batch
    tile = 256
    grid_pre = rows // tile

    xf = input.reshape(rows, ninp).astype(jnp.bfloat16)
    kx_flat, v0_flat = pl.pallas_call(
        _precompute_body,
        grid=(grid_pre,),
        in_specs=[
            pl.BlockSpec((tile, ninp), lambda i: (i, 0)),
            pl.BlockSpec((ninp, _DK), lambda i: (0, 0)),
            pl.BlockSpec((ninp, _DV), lambda i: (0, 0)),
        ],
        out_specs=[
            pl.BlockSpec((tile, _DK), lambda i: (i, 0)),
            pl.BlockSpec((tile, _DV), lambda i: (i, 0)),
        ],
        out_shape=[
            jax.ShapeDtypeStruct((rows, _DK), jnp.bfloat16),
            jax.ShapeDtypeStruct((rows, _DV), jnp.bfloat16),
        ],
    )(xf, Wk_in.astype(jnp.bfloat16), Wv_in.astype(jnp.bfloat16))

    kx_v = kx_flat.reshape(seq, batch, _DK)
    v0_v = v0_flat.reshape(seq, batch, _DV)
    h0 = hidden.reshape(batch, _NB, _BH).transpose(1, 0, 2)

    wq_l = [Wq_in[n] for n in range(_NB)]
    wh_l = [Wh[n] for n in range(_NB)]
    wx_l = [Wx[n] for n in range(_NB)]
    wqc_l = [Wq_c[n] for n in range(_NB)]
    wkc_l = [Wk_c[n] for n in range(_NB)]
    wvc_l = [Wv_c[n] for n in range(_NB)]

    in_specs = [
        pl.BlockSpec((1, batch, _DK), lambda t: (t, 0, 0)),
        pl.BlockSpec((1, batch, _DV), lambda t: (t, 0, 0)),
        _full((_NB, batch, _BH)),
    ]
    in_specs += [_full((_BH, _DK))] * _NB
    in_specs += [_full((_BH, 3 * _BH))] * _NB
    in_specs += [_full((_DV, 3 * _BH))] * _NB
    in_specs += [_full((_BH, _DKC))] * _NB
    in_specs += [_full((_BH, _DKC))] * _NB
    in_specs += [_full((_BH, _BH))] * _NB
    in_specs += [_full((_NB, _DK)), _full((_NB, 3 * _BH))]

    out = pl.pallas_call(
        _step_body,
        grid=(seq,),
        in_specs=in_specs,
        out_specs=pl.BlockSpec((1, batch, _NB * _BH), lambda t: (t, 0, 0)),
        out_shape=jax.ShapeDtypeStruct((seq, batch, _NB * _BH), jnp.float32),
        scratch_shapes=[pltpu.VMEM((_NB, batch, _BH), jnp.float32)],
        compiler_params=pltpu.CompilerParams(
            dimension_semantics=("arbitrary",),
        ),
    )(kx_v, v0_v, h0, *wq_l, *wh_l, *wx_l, *wqc_l, *wkc_l, *wvc_l, bq_in, bg)

    return out


# speedup vs baseline: 1.2673x; 1.0031x over previous
import math

import jax
import jax.numpy as jnp
from jax.experimental import pallas as pl
from jax.experimental.pallas import tpu as pltpu

_NINP = 1024
_NB = 8
_BH = 256
_TOPK = 2
_DK = 64
_DV = 256
_DKC = 32


def _precompute_body(x_ref, wk_ref, wv_ref, kx_ref, v0_ref):
    x = x_ref[...]
    kx = jnp.dot(x, wk_ref[...], preferred_element_type=jnp.float32)
    kx_ref[...] = kx.astype(jnp.bfloat16)
    v0 = jnp.dot(x, wv_ref[...], preferred_element_type=jnp.float32)
    v0_ref[...] = v0.astype(jnp.bfloat16)


def _step_body(*refs):
    (kx_ref, v0_ref, h0_ref) = refs[0:3]
    wqh_refs = refs[3:3 + _NB]
    wx_refs = refs[3 + _NB:3 + 2 * _NB]
    wc_refs = refs[3 + 2 * _NB:3 + 3 * _NB]
    bq_ref, bg_ref = refs[3 + 3 * _NB:5 + 3 * _NB]
    out_ref = refs[5 + 3 * _NB]
    h_scr = refs[6 + 3 * _NB]

    t = pl.program_id(0)

    @pl.when(t == 0)
    def _init():
        h_scr[...] = h0_ref[...]

    kx = kx_ref[0].astype(jnp.float32)
    v0 = v0_ref[0].astype(jnp.float32)

    h_list = [h_scr[n] for n in range(_NB)]
    s_cols = []
    gh_list = []
    for n in range(_NB):
        hq = jnp.dot(h_list[n], wqh_refs[n][...],
                     preferred_element_type=jnp.float32)
        q = hq[:, :_DK] + bq_ref[n]
        qb = q.astype(jnp.bfloat16).astype(jnp.float32)
        s_cols.append(jnp.sum(qb * kx, axis=1, keepdims=True))
        gh_list.append(hq[:, 128:])
    s = jnp.concatenate(s_cols, axis=1) / 8.0

    m = jnp.maximum(s, 0.0)
    e = jnp.exp(s - m)
    att0 = e / (e + jnp.exp(-m))
    attb = att0.astype(jnp.bfloat16).astype(jnp.float32)

    hn_list = []
    for n in range(_NB):
        xin = attb[:, n:n + 1] * v0
        gx = jnp.dot(xin, wx_refs[n][...],
                     preferred_element_type=jnp.float32) + bg_ref[n]
        gh = gh_list[n]
        r = jax.nn.sigmoid(gx[:, :_BH] + gh[:, :_BH])
        z = jax.nn.sigmoid(gx[:, _BH:2 * _BH] + gh[:, _BH:2 * _BH])
        g = jnp.tanh(gx[:, 2 * _BH:] + r * gh[:, 2 * _BH:])
        hn_list.append((1.0 - z) * g + z * h_list[n])

    qc_list, kc_list, vc_list = [], [], []
    for n in range(_NB):
        c = jnp.dot(hn_list[n], wc_refs[n][...],
                    preferred_element_type=jnp.float32)
        qc_list.append(c[:, :_DKC].astype(jnp.bfloat16).astype(jnp.float32))
        kc_list.append(c[:, 128:128 + _DKC].astype(jnp.bfloat16).astype(jnp.float32))
        vc_list.append(c[:, 256:])
    qcs = jnp.stack(qc_list)
    kcs = jnp.stack(kc_list)
    vcs = jnp.stack(vc_list)
    logits = (jnp.sum(qcs[:, None] * kcs[None, :], axis=-1)
              / math.sqrt(_DKC))
    lmax = jnp.max(logits, axis=1, keepdims=True)
    le = jnp.exp(logits - lmax)
    ac = le / jnp.sum(le, axis=1, keepdims=True)
    acb = ac.astype(jnp.bfloat16).astype(jnp.float32)
    vcb = vcs.astype(jnp.bfloat16).astype(jnp.float32)

    r1 = att0[:, None, :]
    r2 = att0[:, :, None]
    n_idx = jax.lax.broadcasted_iota(jnp.int32, (1, _NB, _NB), 1)
    m_idx = jax.lax.broadcasted_iota(jnp.int32, (1, _NB, _NB), 2)
    beats = (r1 > r2) | ((r1 == r2) & (m_idx < n_idx))
    rank = jnp.sum(beats.astype(jnp.int32), axis=2)
    maskf = (rank < _TOPK).astype(jnp.float32)

    for n in range(_NB):
        comm = jnp.sum(acb[n][:, :, None] * vcb, axis=0)
        hn2 = hn_list[n] + comm
        mk = maskf[:, n:n + 1]
        hout = mk * hn2 + (1.0 - mk) * h_list[n]
        h_scr[n] = hout
        out_ref[0, :, n * _BH:(n + 1) * _BH] = hout


def _full(shape):
    nd = len(shape)
    return pl.BlockSpec(shape, lambda t, _nd=nd: (0,) * _nd)


def kernel(input, hidden, seq_len, Wq_in, bq_in, Wk_in, Wv_in, Wx, Wh, bg,
           Wq_c, Wk_c, Wv_c):
    seq, batch, ninp = input.shape
    rows = seq * batch
    tile = 256
    grid_pre = rows // tile

    xf = input.reshape(rows, ninp).astype(jnp.bfloat16)
    kx_flat, v0_flat = pl.pallas_call(
        _precompute_body,
        grid=(grid_pre,),
        in_specs=[
            pl.BlockSpec((tile, ninp), lambda i: (i, 0)),
            pl.BlockSpec((ninp, _DK), lambda i: (0, 0)),
            pl.BlockSpec((ninp, _DV), lambda i: (0, 0)),
        ],
        out_specs=[
            pl.BlockSpec((tile, _DK), lambda i: (i, 0)),
            pl.BlockSpec((tile, _DV), lambda i: (i, 0)),
        ],
        out_shape=[
            jax.ShapeDtypeStruct((rows, _DK), jnp.bfloat16),
            jax.ShapeDtypeStruct((rows, _DV), jnp.bfloat16),
        ],
    )(xf, Wk_in.astype(jnp.bfloat16), Wv_in.astype(jnp.bfloat16))

    kx_v = kx_flat.reshape(seq, batch, _DK)
    v0_v = v0_flat.reshape(seq, batch, _DV)
    h0 = hidden.reshape(batch, _NB, _BH).transpose(1, 0, 2)

    zq = jnp.zeros((_NB, _BH, 64), jnp.float32)
    wqh = jnp.concatenate([Wq_in, zq, Wh], axis=2)
    zc = jnp.zeros((_NB, _BH, 96), jnp.float32)
    wcp = jnp.concatenate([Wq_c, zc, Wk_c, zc, Wv_c], axis=2)

    wqh_l = [wqh[n] for n in range(_NB)]
    wx_l = [Wx[n] for n in range(_NB)]
    wc_l = [wcp[n] for n in range(_NB)]

    in_specs = [
        pl.BlockSpec((1, batch, _DK), lambda t: (t, 0, 0)),
        pl.BlockSpec((1, batch, _DV), lambda t: (t, 0, 0)),
        _full((_NB, batch, _BH)),
    ]
    in_specs += [_full((_BH, 896))] * _NB
    in_specs += [_full((_DV, 3 * _BH))] * _NB
    in_specs += [_full((_BH, 512))] * _NB
    in_specs += [_full((_NB, _DK)), _full((_NB, 3 * _BH))]

    out = pl.pallas_call(
        _step_body,
        grid=(seq,),
        in_specs=in_specs,
        out_specs=pl.BlockSpec((1, batch, _NB * _BH), lambda t: (t, 0, 0)),
        out_shape=jax.ShapeDtypeStruct((seq, batch, _NB * _BH), jnp.float32),
        scratch_shapes=[pltpu.VMEM((_NB, batch, _BH), jnp.float32)],
        compiler_params=pltpu.CompilerParams(
            dimension_semantics=("arbitrary",),
        ),
    )(kx_v, v0_v, h0, *wqh_l, *wx_l, *wc_l, bq_in, bg)

    return out


# speedup vs baseline: 9.6995x; 7.6535x over previous
import math

import jax
import jax.numpy as jnp
from jax.experimental import pallas as pl
from jax.experimental.pallas import tpu as pltpu

_NINP = 1024
_NB = 8
_BH = 256
_TOPK = 2
_DK = 64
_DV = 256
_DKC = 32

_DN00 = (((0,), (0,)), ((), ()))


def _precompute_body(x_ref, wk_ref, wv_ref, kx_ref, v0_ref):
    x = x_ref[...]
    kx = jnp.dot(x, wk_ref[...], preferred_element_type=jnp.float32)
    kx_ref[...] = kx.astype(jnp.bfloat16)
    v0 = jnp.dot(x, wv_ref[...], preferred_element_type=jnp.float32)
    v0_ref[...] = v0.astype(jnp.bfloat16)


def _bf(x):
    return x.astype(jnp.bfloat16).astype(jnp.float32)


def _step_body(*refs):
    (kx_ref, v0_ref, h0t_ref) = refs[0:3]
    wqh_refs = refs[3:3 + _NB]
    wx_refs = refs[3 + _NB:3 + 2 * _NB]
    wc_refs = refs[3 + 2 * _NB:3 + 3 * _NB]
    bqt_ref, bgt_ref = refs[3 + 3 * _NB:5 + 3 * _NB]
    i64b_ref, i64f_ref, i256f_ref = refs[5 + 3 * _NB:8 + 3 * _NB]
    out_ref = refs[8 + 3 * _NB]
    ht_scr = refs[9 + 3 * _NB]

    t = pl.program_id(0)

    @pl.when(t == 0)
    def _init():
        ht_scr[...] = h0t_ref[...]

    kx = kx_ref[0].astype(jnp.float32)
    v0t = jax.lax.dot_general(v0_ref[0], i64b_ref[...], _DN00,
                              preferred_element_type=jnp.float32)

    ht_list = [ht_scr[n] for n in range(_NB)]
    s_cols = []
    ght_list = []
    for n in range(_NB):
        hqt = jax.lax.dot_general(wqh_refs[n][...], ht_list[n], _DN00,
                                  preferred_element_type=jnp.float32)
        qt = hqt[:_DK] + bqt_ref[n]
        ght_list.append(hqt[128:])
        qr = jax.lax.dot_general(qt, i64f_ref[...], _DN00,
                                 preferred_element_type=jnp.float32)
        s_cols.append(jnp.sum(qr * kx, axis=1, keepdims=True))
    s = jnp.concatenate(s_cols, axis=1) / 8.0

    m = jnp.maximum(s, 0.0)
    e = jnp.exp(s - m)
    att0 = e / (e + jnp.exp(-m))
    attb = _bf(att0)
    attt = jax.lax.dot_general(attb, i64f_ref[...], _DN00,
                               preferred_element_type=jnp.float32)

    hnt_list = []
    for n in range(_NB):
        xint = attt[n:n + 1, :] * v0t
        gxt = jax.lax.dot_general(wx_refs[n][...], xint, _DN00,
                                  preferred_element_type=jnp.float32)
        gxt = gxt + bgt_ref[n]
        ght = ght_list[n]
        r = jax.nn.sigmoid(gxt[:_BH] + ght[:_BH])
        z = jax.nn.sigmoid(gxt[_BH:2 * _BH] + ght[_BH:2 * _BH])
        g = jnp.tanh(gxt[2 * _BH:] + r * ght[2 * _BH:])
        hnt_list.append((1.0 - z) * g + z * ht_list[n])

    qct_list, kct_list, vct_list = [], [], []
    for n in range(_NB):
        ct = jax.lax.dot_general(wc_refs[n][...], hnt_list[n], _DN00,
                                 preferred_element_type=jnp.float32)
        qct_list.append(_bf(ct[:_DKC]))
        kct_list.append(_bf(ct[128:128 + _DKC]))
        vct_list.append(ct[256:])
    qcst = jnp.stack(qct_list)
    kcst = jnp.stack(kct_list)
    vcst = jnp.stack(vct_list)
    logits = (jnp.sum(qcst[:, None] * kcst[None, :], axis=2)
              / math.sqrt(_DKC))
    lmax = jnp.max(logits, axis=1, keepdims=True)
    le = jnp.exp(logits - lmax)
    ac = le / jnp.sum(le, axis=1, keepdims=True)
    acb = _bf(ac)
    vcbt = _bf(vcst)

    r1 = att0[:, None, :]
    r2 = att0[:, :, None]
    n_idx = jax.lax.broadcasted_iota(jnp.int32, (1, _NB, _NB), 1)
    m_idx = jax.lax.broadcasted_iota(jnp.int32, (1, _NB, _NB), 2)
    beats = (r1 > r2) | ((r1 == r2) & (m_idx < n_idx))
    rank = jnp.sum(beats.astype(jnp.int32), axis=2)
    maskf = (rank < _TOPK).astype(jnp.float32)
    maskt = jax.lax.dot_general(maskf, i64f_ref[...], _DN00,
                                preferred_element_type=jnp.float32)

    for n in range(_NB):
        commt = jnp.sum(acb[n][:, None, :] * vcbt, axis=0)
        hn2t = hnt_list[n] + commt
        mkt = maskt[n:n + 1, :]
        houtt = mkt * hn2t + (1.0 - mkt) * ht_list[n]
        ht_scr[n] = houtt
        hout = jax.lax.dot_general(houtt, i256f_ref[...], _DN00,
                                   preferred_element_type=jnp.float32,
                                   precision=jax.lax.Precision.HIGHEST)
        out_ref[0, :, n * _BH:(n + 1) * _BH] = hout


def _full(shape):
    nd = len(shape)
    return pl.BlockSpec(shape, lambda t, _nd=nd: (0,) * _nd)


def kernel(input, hidden, seq_len, Wq_in, bq_in, Wk_in, Wv_in, Wx, Wh, bg,
           Wq_c, Wk_c, Wv_c):
    seq, batch, ninp = input.shape
    rows = seq * batch
    tile = 256
    grid_pre = rows // tile

    xf = input.reshape(rows, ninp).astype(jnp.bfloat16)
    kx_flat, v0_flat = pl.pallas_call(
        _precompute_body,
        grid=(grid_pre,),
        in_specs=[
            pl.BlockSpec((tile, ninp), lambda i: (i, 0)),
            pl.BlockSpec((ninp, _DK), lambda i: (0, 0)),
            pl.BlockSpec((ninp, _DV), lambda i: (0, 0)),
        ],
        out_specs=[
            pl.BlockSpec((tile, _DK), lambda i: (i, 0)),
            pl.BlockSpec((tile, _DV), lambda i: (i, 0)),
        ],
        out_shape=[
            jax.ShapeDtypeStruct((rows, _DK), jnp.bfloat16),
            jax.ShapeDtypeStruct((rows, _DV), jnp.bfloat16),
        ],
    )(xf, Wk_in.astype(jnp.bfloat16), Wv_in.astype(jnp.bfloat16))

    kx_v = kx_flat.reshape(seq, batch, _DK)
    v0_v = v0_flat.reshape(seq, batch, _DV)
    h0t = hidden.reshape(batch, _NB, _BH).transpose(1, 2, 0)

    zq = jnp.zeros((_NB, _BH, 64), jnp.float32)
    wqh = jnp.concatenate([Wq_in, zq, Wh], axis=2)
    zc = jnp.zeros((_NB, _BH, 96), jnp.float32)
    wcp = jnp.concatenate([Wq_c, zc, Wk_c, zc, Wv_c], axis=2)

    wqh_l = [wqh[n] for n in range(_NB)]
    wx_l = [Wx[n] for n in range(_NB)]
    wc_l = [wcp[n] for n in range(_NB)]

    bqt = jnp.broadcast_to(bq_in[:, :, None], (_NB, _DK, batch))
    bgt = jnp.broadcast_to(bg[:, :, None], (_NB, 3 * _BH, batch))
    i64b = jnp.eye(batch, dtype=jnp.bfloat16)
    i64f = jnp.eye(batch, dtype=jnp.float32)
    i256f = jnp.eye(_BH, dtype=jnp.float32)

    in_specs = [
        pl.BlockSpec((1, batch, _DK), lambda t: (t, 0, 0)),
        pl.BlockSpec((1, batch, _DV), lambda t: (t, 0, 0)),
        _full((_NB, _BH, batch)),
    ]
    in_specs += [_full((_BH, 896))] * _NB
    in_specs += [_full((_DV, 3 * _BH))] * _NB
    in_specs += [_full((_BH, 512))] * _NB
    in_specs += [_full((_NB, _DK, batch)), _full((_NB, 3 * _BH, batch))]
    in_specs += [_full((batch, batch)), _full((batch, batch)),
                 _full((_BH, _BH))]

    out = pl.pallas_call(
        _step_body,
        grid=(seq,),
        in_specs=in_specs,
        out_specs=pl.BlockSpec((1, batch, _NB * _BH), lambda t: (t, 0, 0)),
        out_shape=jax.ShapeDtypeStruct((seq, batch, _NB * _BH), jnp.float32),
        scratch_shapes=[pltpu.VMEM((_NB, _BH, batch), jnp.float32)],
        compiler_params=pltpu.CompilerParams(
            dimension_semantics=("arbitrary",),
        ),
    )(kx_v, v0_v, h0t, *wqh_l, *wx_l, *wc_l, bqt, bgt, i64b, i64f, i256f)

    return out


# speedup vs baseline: 10.6832x; 1.1014x over previous
import math

import jax
import jax.numpy as jnp
from jax.experimental import pallas as pl
from jax.experimental.pallas import tpu as pltpu

_NINP = 1024
_NB = 8
_BH = 256
_TOPK = 2
_DK = 64
_DV = 256
_DKC = 32

_DN00 = (((0,), (0,)), ((), ()))


def _precompute_body(x_ref, wk_ref, wv_ref, kx_ref, v0_ref):
    x = x_ref[...]
    kx = jnp.dot(x, wk_ref[...], preferred_element_type=jnp.float32)
    kx_ref[...] = kx.astype(jnp.bfloat16)
    v0 = jnp.dot(x, wv_ref[...], preferred_element_type=jnp.float32)
    v0_ref[...] = v0.astype(jnp.bfloat16)


def _bf(x):
    return x.astype(jnp.bfloat16).astype(jnp.float32)


def _step_body(*refs):
    (kx_ref, v0_ref, h0t_ref) = refs[0:3]
    wqh_refs = refs[3:3 + _NB]
    wx_refs = refs[3 + _NB:3 + 2 * _NB]
    wc_refs = refs[3 + 2 * _NB:3 + 3 * _NB]
    bqt_ref, bgt_ref = refs[3 + 3 * _NB:5 + 3 * _NB]
    i64b_ref, i64f_ref, i256f_ref = refs[5 + 3 * _NB:8 + 3 * _NB]
    out_ref = refs[8 + 3 * _NB]
    ht_scr = refs[9 + 3 * _NB]

    t = pl.program_id(0)

    @pl.when(t == 0)
    def _init():
        ht_scr[...] = h0t_ref[...]

    kx = kx_ref[0].astype(jnp.float32)
    v0t = jax.lax.dot_general(v0_ref[0], i64b_ref[...], _DN00,
                              preferred_element_type=jnp.float32)

    ht_list = [ht_scr[n] for n in range(_NB)]
    s_cols = []
    ght_list = []
    for n in range(_NB):
        hqt = jax.lax.dot_general(wqh_refs[n][...], ht_list[n], _DN00,
                                  preferred_element_type=jnp.float32)
        qt = hqt[:_DK] + bqt_ref[n]
        ght_list.append(hqt[_DK:])
        qr = jax.lax.dot_general(qt, i64f_ref[...], _DN00,
                                 preferred_element_type=jnp.float32)
        s_cols.append(jnp.sum(qr * kx, axis=1, keepdims=True))
    s = jnp.concatenate(s_cols, axis=1) / 8.0

    m = jnp.maximum(s, 0.0)
    e = jnp.exp(s - m)
    att0 = e / (e + jnp.exp(-m))
    attb = _bf(att0)
    attt = jax.lax.dot_general(attb, i64f_ref[...], _DN00,
                               preferred_element_type=jnp.float32)

    hnt_list = []
    for n in range(_NB):
        xint = attt[n:n + 1, :] * v0t
        gxt = jax.lax.dot_general(wx_refs[n][...], xint, _DN00,
                                  preferred_element_type=jnp.float32)
        gxt = gxt + bgt_ref[n]
        ght = ght_list[n]
        r = jax.nn.sigmoid(gxt[:_BH] + ght[:_BH])
        z = jax.nn.sigmoid(gxt[_BH:2 * _BH] + ght[_BH:2 * _BH])
        g = jnp.tanh(gxt[2 * _BH:] + r * ght[2 * _BH:])
        hnt_list.append((1.0 - z) * g + z * ht_list[n])

    qct_list, kct_list, vct_list = [], [], []
    for n in range(_NB):
        ct = jax.lax.dot_general(wc_refs[n][...], hnt_list[n], _DN00,
                                 preferred_element_type=jnp.float32)
        qct_list.append(_bf(ct[:_DKC]))
        kct_list.append(_bf(ct[_DKC:2 * _DKC]))
        vct_list.append(ct[2 * _DKC:])
    qcst = jnp.stack(qct_list)
    kcst = jnp.stack(kct_list)
    vcst = jnp.stack(vct_list)
    logits = (jnp.sum(qcst[:, None] * kcst[None, :], axis=2)
              / math.sqrt(_DKC))
    lmax = jnp.max(logits, axis=1, keepdims=True)
    le = jnp.exp(logits - lmax)
    ac = le / jnp.sum(le, axis=1, keepdims=True)
    acb = _bf(ac)
    vcbt = _bf(vcst)

    r1 = att0[:, None, :]
    r2 = att0[:, :, None]
    n_idx = jax.lax.broadcasted_iota(jnp.int32, (1, _NB, _NB), 1)
    m_idx = jax.lax.broadcasted_iota(jnp.int32, (1, _NB, _NB), 2)
    beats = (r1 > r2) | ((r1 == r2) & (m_idx < n_idx))
    rank = jnp.sum(beats.astype(jnp.int32), axis=2)
    maskf = (rank < _TOPK).astype(jnp.float32)
    maskt = jax.lax.dot_general(maskf, i64f_ref[...], _DN00,
                                preferred_element_type=jnp.float32)

    for n in range(_NB):
        commt = jnp.sum(acb[n][:, None, :] * vcbt, axis=0)
        hn2t = hnt_list[n] + commt
        mkt = maskt[n:n + 1, :]
        houtt = mkt * hn2t + (1.0 - mkt) * ht_list[n]
        ht_scr[n] = houtt
        hout = jax.lax.dot_general(houtt, i256f_ref[...], _DN00,
                                   preferred_element_type=jnp.float32,
                                   precision=jax.lax.Precision.HIGHEST)
        out_ref[0, :, n * _BH:(n + 1) * _BH] = hout


def _full(shape):
    nd = len(shape)
    return pl.BlockSpec(shape, lambda t, _nd=nd: (0,) * _nd)


def kernel(input, hidden, seq_len, Wq_in, bq_in, Wk_in, Wv_in, Wx, Wh, bg,
           Wq_c, Wk_c, Wv_c):
    seq, batch, ninp = input.shape
    rows = seq * batch
    tile = 256
    grid_pre = rows // tile

    xf = input.reshape(rows, ninp).astype(jnp.bfloat16)
    kx_flat, v0_flat = pl.pallas_call(
        _precompute_body,
        grid=(grid_pre,),
        in_specs=[
            pl.BlockSpec((tile, ninp), lambda i: (i, 0)),
            pl.BlockSpec((ninp, _DK), lambda i: (0, 0)),
            pl.BlockSpec((ninp, _DV), lambda i: (0, 0)),
        ],
        out_specs=[
            pl.BlockSpec((tile, _DK), lambda i: (i, 0)),
            pl.BlockSpec((tile, _DV), lambda i: (i, 0)),
        ],
        out_shape=[
            jax.ShapeDtypeStruct((rows, _DK), jnp.bfloat16),
            jax.ShapeDtypeStruct((rows, _DV), jnp.bfloat16),
        ],
    )(xf, Wk_in.astype(jnp.bfloat16), Wv_in.astype(jnp.bfloat16))

    kx_v = kx_flat.reshape(seq, batch, _DK)
    v0_v = v0_flat.reshape(seq, batch, _DV)
    h0t = hidden.reshape(batch, _NB, _BH).transpose(1, 2, 0)

    wqh = jnp.concatenate([Wq_in, Wh], axis=2)
    wcp = jnp.concatenate([Wq_c, Wk_c, Wv_c], axis=2)

    wqh_l = [wqh[n] for n in range(_NB)]
    wx_l = [Wx[n] for n in range(_NB)]
    wc_l = [wcp[n] for n in range(_NB)]

    bqt = jnp.broadcast_to(bq_in[:, :, None], (_NB, _DK, batch))
    bgt = jnp.broadcast_to(bg[:, :, None], (_NB, 3 * _BH, batch))
    i64b = jnp.eye(batch, dtype=jnp.bfloat16)
    i64f = jnp.eye(batch, dtype=jnp.float32)
    i256f = jnp.eye(_BH, dtype=jnp.float32)

    in_specs = [
        pl.BlockSpec((1, batch, _DK), lambda t: (t, 0, 0)),
        pl.BlockSpec((1, batch, _DV), lambda t: (t, 0, 0)),
        _full((_NB, _BH, batch)),
    ]
    in_specs += [_full((_BH, _DK + 3 * _BH))] * _NB
    in_specs += [_full((_DV, 3 * _BH))] * _NB
    in_specs += [_full((_BH, 2 * _DKC + _BH))] * _NB
    in_specs += [_full((_NB, _DK, batch)), _full((_NB, 3 * _BH, batch))]
    in_specs += [_full((batch, batch)), _full((batch, batch)),
                 _full((_BH, _BH))]

    out = pl.pallas_call(
        _step_body,
        grid=(seq,),
        in_specs=in_specs,
        out_specs=pl.BlockSpec((1, batch, _NB * _BH), lambda t: (t, 0, 0)),
        out_shape=jax.ShapeDtypeStruct((seq, batch, _NB * _BH), jnp.float32),
        scratch_shapes=[pltpu.VMEM((_NB, _BH, batch), jnp.float32)],
        compiler_params=pltpu.CompilerParams(
            dimension_semantics=("arbitrary",),
        ),
    )(kx_v, v0_v, h0t, *wqh_l, *wx_l, *wc_l, bqt, bgt, i64b, i64f, i256f)

    return out


# speedup vs baseline: 11.2867x; 1.0565x over previous
import math

import jax
import jax.numpy as jnp
from jax.experimental import pallas as pl
from jax.experimental.pallas import tpu as pltpu

_NINP = 1024
_NB = 8
_BH = 256
_TOPK = 2
_DK = 64
_DV = 256
_DKC = 32

_DN00 = (((0,), (0,)), ((), ()))


def _precompute_body(x_ref, wk_ref, wv_ref, kx_ref, v0_ref):
    x = x_ref[...]
    kx = jnp.dot(x, wk_ref[...], preferred_element_type=jnp.float32)
    kx_ref[...] = kx.astype(jnp.bfloat16)
    v0 = jnp.dot(x, wv_ref[...], preferred_element_type=jnp.float32)
    v0_ref[...] = v0.astype(jnp.bfloat16)


def _bf(x):
    return x.astype(jnp.bfloat16).astype(jnp.float32)


def _step_body(*refs):
    (kx_ref, v0_ref, h0t_ref) = refs[0:3]
    wq_refs = refs[3:3 + _NB]
    wh_refs = refs[3 + _NB:3 + 2 * _NB]
    wx_refs = refs[3 + 2 * _NB:3 + 3 * _NB]
    wqc_refs = refs[3 + 3 * _NB:3 + 4 * _NB]
    wkc_refs = refs[3 + 4 * _NB:3 + 5 * _NB]
    wvc_refs = refs[3 + 5 * _NB:3 + 6 * _NB]
    bqt_ref, bgt_ref = refs[3 + 6 * _NB:5 + 6 * _NB]
    i64b_ref, i256f_ref = refs[5 + 6 * _NB:7 + 6 * _NB]
    out_ref = refs[7 + 6 * _NB]
    ht_scr = refs[8 + 6 * _NB]

    t = pl.program_id(0)

    @pl.when(t == 0)
    def _init():
        ht_scr[...] = h0t_ref[...]

    kxt = jax.lax.dot_general(kx_ref[0], i64b_ref[...], _DN00,
                              preferred_element_type=jnp.float32)
    v0t = jax.lax.dot_general(v0_ref[0], i64b_ref[...], _DN00,
                              preferred_element_type=jnp.float32)

    ht_list = [ht_scr[n] for n in range(_NB)]
    s_rows = []
    ght_list = []
    for n in range(_NB):
        qt = jax.lax.dot_general(wq_refs[n][...], ht_list[n], _DN00,
                                 preferred_element_type=jnp.float32)
        qbt = _bf(qt + bqt_ref[n])
        s_rows.append(jnp.sum(qbt * kxt, axis=0, keepdims=True))
        ght_list.append(jax.lax.dot_general(wh_refs[n][...], ht_list[n],
                                            _DN00,
                                            preferred_element_type=jnp.float32))
    st = jnp.concatenate(s_rows, axis=0) / 8.0

    m = jnp.maximum(st, 0.0)
    e = jnp.exp(st - m)
    att0t = e / (e + jnp.exp(-m))
    attbt = _bf(att0t)

    hnt_list = []
    for n in range(_NB):
        xint = attbt[n:n + 1, :] * v0t
        gxt = jax.lax.dot_general(wx_refs[n][...], xint, _DN00,
                                  preferred_element_type=jnp.float32)
        gxt = gxt + bgt_ref[n]
        ght = ght_list[n]
        r = jax.nn.sigmoid(gxt[:_BH] + ght[:_BH])
        z = jax.nn.sigmoid(gxt[_BH:2 * _BH] + ght[_BH:2 * _BH])
        g = jnp.tanh(gxt[2 * _BH:] + r * ght[2 * _BH:])
        hnt_list.append((1.0 - z) * g + z * ht_list[n])

    qct_list, kct_list, vct_list = [], [], []
    for n in range(_NB):
        qct = jax.lax.dot_general(wqc_refs[n][...], hnt_list[n], _DN00,
                                  preferred_element_type=jnp.float32)
        kct = jax.lax.dot_general(wkc_refs[n][...], hnt_list[n], _DN00,
                                  preferred_element_type=jnp.float32)
        vct = jax.lax.dot_general(wvc_refs[n][...], hnt_list[n], _DN00,
                                  preferred_element_type=jnp.float32)
        qct_list.append(_bf(qct))
        kct_list.append(_bf(kct))
        vct_list.append(vct)
    qcst = jnp.stack(qct_list)
    kcst = jnp.stack(kct_list)
    vcst = jnp.stack(vct_list)
    logits = (jnp.sum(qcst[:, None] * kcst[None, :], axis=2)
              / math.sqrt(_DKC))
    lmax = jnp.max(logits, axis=1, keepdims=True)
    le = jnp.exp(logits - lmax)
    ac = le / jnp.sum(le, axis=1, keepdims=True)
    acb = _bf(ac)
    vcbt = _bf(vcst)

    rn = att0t[:, None, :]
    rm = att0t[None, :, :]
    n_idx = jax.lax.broadcasted_iota(jnp.int32, (_NB, _NB, 1), 0)
    m_idx = jax.lax.broadcasted_iota(jnp.int32, (_NB, _NB, 1), 1)
    beats = (rm > rn) | ((rm == rn) & (m_idx < n_idx))
    rank = jnp.sum(beats.astype(jnp.int32), axis=1)
    maskt = (rank < _TOPK).astype(jnp.float32)

    for n in range(_NB):
        commt = jnp.sum(acb[n][:, None, :] * vcbt, axis=0)
        hn2t = hnt_list[n] + commt
        mkt = maskt[n:n + 1, :]
        houtt = mkt * hn2t + (1.0 - mkt) * ht_list[n]
        ht_scr[n] = houtt
        hout = jax.lax.dot_general(houtt, i256f_ref[...], _DN00,
                                   preferred_element_type=jnp.float32,
                                   precision=jax.lax.Precision.HIGHEST)
        out_ref[0, :, n * _BH:(n + 1) * _BH] = hout


def _full(shape):
    nd = len(shape)
    return pl.BlockSpec(shape, lambda t, _nd=nd: (0,) * _nd)


def kernel(input, hidden, seq_len, Wq_in, bq_in, Wk_in, Wv_in, Wx, Wh, bg,
           Wq_c, Wk_c, Wv_c):
    seq, batch, ninp = input.shape
    rows = seq * batch
    tile = 256
    grid_pre = rows // tile

    xf = input.reshape(rows, ninp).astype(jnp.bfloat16)
    kx_flat, v0_flat = pl.pallas_call(
        _precompute_body,
        grid=(grid_pre,),
        in_specs=[
            pl.BlockSpec((tile, ninp), lambda i: (i, 0)),
            pl.BlockSpec((ninp, _DK), lambda i: (0, 0)),
            pl.BlockSpec((ninp, _DV), lambda i: (0, 0)),
        ],
        out_specs=[
            pl.BlockSpec((tile, _DK), lambda i: (i, 0)),
            pl.BlockSpec((tile, _DV), lambda i: (i, 0)),
        ],
        out_shape=[
            jax.ShapeDtypeStruct((rows, _DK), jnp.bfloat16),
            jax.ShapeDtypeStruct((rows, _DV), jnp.bfloat16),
        ],
    )(xf, Wk_in.astype(jnp.bfloat16), Wv_in.astype(jnp.bfloat16))

    kx_v = kx_flat.reshape(seq, batch, _DK)
    v0_v = v0_flat.reshape(seq, batch, _DV)
    h0t = hidden.reshape(batch, _NB, _BH).transpose(1, 2, 0)

    wq_l = [Wq_in[n] for n in range(_NB)]
    wh_l = [Wh[n] for n in range(_NB)]
    wx_l = [Wx[n] for n in range(_NB)]
    wqc_l = [Wq_c[n] for n in range(_NB)]
    wkc_l = [Wk_c[n] for n in range(_NB)]
    wvc_l = [Wv_c[n] for n in range(_NB)]

    bqt = jnp.broadcast_to(bq_in[:, :, None], (_NB, _DK, batch))
    bgt = jnp.broadcast_to(bg[:, :, None], (_NB, 3 * _BH, batch))
    i64b = jnp.eye(batch, dtype=jnp.bfloat16)
    i256f = jnp.eye(_BH, dtype=jnp.float32)

    in_specs = [
        pl.BlockSpec((1, batch, _DK), lambda t: (t, 0, 0)),
        pl.BlockSpec((1, batch, _DV), lambda t: (t, 0, 0)),
        _full((_NB, _BH, batch)),
    ]
    in_specs += [_full((_BH, _DK))] * _NB
    in_specs += [_full((_BH, 3 * _BH))] * _NB
    in_specs += [_full((_DV, 3 * _BH))] * _NB
    in_specs += [_full((_BH, _DKC))] * _NB
    in_specs += [_full((_BH, _DKC))] * _NB
    in_specs += [_full((_BH, _BH))] * _NB
    in_specs += [_full((_NB, _DK, batch)), _full((_NB, 3 * _BH, batch))]
    in_specs += [_full((batch, batch)), _full((_BH, _BH))]

    out = pl.pallas_call(
        _step_body,
        grid=(seq,),
        in_specs=in_specs,
        out_specs=pl.BlockSpec((1, batch, _NB * _BH), lambda t: (t, 0, 0)),
        out_shape=jax.ShapeDtypeStruct((seq, batch, _NB * _BH), jnp.float32),
        scratch_shapes=[pltpu.VMEM((_NB, _BH, batch), jnp.float32)],
        compiler_params=pltpu.CompilerParams(
            dimension_semantics=("arbitrary",),
        ),
    )(kx_v, v0_v, h0t, *wq_l, *wh_l, *wx_l, *wqc_l, *wkc_l, *wvc_l,
      bqt, bgt, i64b, i256f)

    return out


# speedup vs baseline: 12.7224x; 1.1272x over previous
import math

import jax
import jax.numpy as jnp
from jax.experimental import pallas as pl
from jax.experimental.pallas import tpu as pltpu

_NINP = 1024
_NB = 8
_BH = 256
_TOPK = 2
_DK = 64
_DV = 256
_DKC = 32

_DN00 = (((0,), (0,)), ((), ()))


def _precompute_body(x_ref, wk_ref, wv_ref, kx_ref, v0_ref):
    x = x_ref[...]
    kx = jnp.dot(x, wk_ref[...], preferred_element_type=jnp.float32)
    kx_ref[...] = kx.astype(jnp.bfloat16)
    v0 = jnp.dot(x, wv_ref[...], preferred_element_type=jnp.float32)
    v0_ref[...] = v0.astype(jnp.bfloat16)


def _bf(x):
    return x.astype(jnp.bfloat16).astype(jnp.float32)


def _step_body(*refs):
    (kx_ref, v0_ref, h0t_ref) = refs[0:3]
    wq_ref, wh_ref, wx_ref, wqc_ref, wkc_ref, wvc_ref = refs[3:9]
    bqt_ref, bgt_ref = refs[9:11]
    i64b_ref, i256f_ref = refs[11:13]
    out_ref = refs[13]
    ht_scr = refs[14]

    t = pl.program_id(0)

    @pl.when(t == 0)
    def _init():
        ht_scr[...] = h0t_ref[...]

    kxt = jax.lax.dot_general(kx_ref[0], i64b_ref[...], _DN00,
                              preferred_element_type=jnp.float32)
    v0t = jax.lax.dot_general(v0_ref[0], i64b_ref[...], _DN00,
                              preferred_element_type=jnp.float32)

    ht_list = [ht_scr[n] for n in range(_NB)]
    s_rows = []
    ght_list = []
    for n in range(_NB):
        qt = jax.lax.dot_general(wq_ref[n], ht_list[n], _DN00,
                                 preferred_element_type=jnp.float32)
        qbt = _bf(qt + bqt_ref[n])
        s_rows.append(jnp.sum(qbt * kxt, axis=0, keepdims=True))
        ght_list.append(jax.lax.dot_general(wh_ref[n], ht_list[n],
                                            _DN00,
                                            preferred_element_type=jnp.float32))
    st = jnp.concatenate(s_rows, axis=0) / 8.0

    m = jnp.maximum(st, 0.0)
    e = jnp.exp(st - m)
    att0t = e / (e + jnp.exp(-m))
    attbt = _bf(att0t)

    hnt_list = []
    for n in range(_NB):
        xint = attbt[n:n + 1, :] * v0t
        gxt = jax.lax.dot_general(wx_ref[n], xint, _DN00,
                                  preferred_element_type=jnp.float32)
        gxt = gxt + bgt_ref[n]
        ght = ght_list[n]
        r = jax.nn.sigmoid(gxt[:_BH] + ght[:_BH])
        z = jax.nn.sigmoid(gxt[_BH:2 * _BH] + ght[_BH:2 * _BH])
        g = jnp.tanh(gxt[2 * _BH:] + r * ght[2 * _BH:])
        hnt_list.append((1.0 - z) * g + z * ht_list[n])

    qct_list, kct_list, vct_list = [], [], []
    for n in range(_NB):
        qct = jax.lax.dot_general(wqc_ref[n], hnt_list[n], _DN00,
                                  preferred_element_type=jnp.float32)
        kct = jax.lax.dot_general(wkc_ref[n], hnt_list[n], _DN00,
                                  preferred_element_type=jnp.float32)
        vct = jax.lax.dot_general(wvc_ref[n], hnt_list[n], _DN00,
                                  preferred_element_type=jnp.float32)
        qct_list.append(_bf(qct))
        kct_list.append(_bf(kct))
        vct_list.append(vct)
    qcst = jnp.stack(qct_list)
    kcst = jnp.stack(kct_list)
    vcst = jnp.stack(vct_list)
    logits = (jnp.sum(qcst[:, None] * kcst[None, :], axis=2)
              / math.sqrt(_DKC))
    lmax = jnp.max(logits, axis=1, keepdims=True)
    le = jnp.exp(logits - lmax)
    ac = le / jnp.sum(le, axis=1, keepdims=True)
    acb = _bf(ac)
    vcbt = _bf(vcst)

    rn = att0t[:, None, :]
    rm = att0t[None, :, :]
    n_idx = jax.lax.broadcasted_iota(jnp.int32, (_NB, _NB, 1), 0)
    m_idx = jax.lax.broadcasted_iota(jnp.int32, (_NB, _NB, 1), 1)
    beats = (rm > rn) | ((rm == rn) & (m_idx < n_idx))
    rank = jnp.sum(beats.astype(jnp.int32), axis=1)
    maskt = (rank < _TOPK).astype(jnp.float32)

    for n in range(_NB):
        commt = jnp.sum(acb[n][:, None, :] * vcbt, axis=0)
        hn2t = hnt_list[n] + commt
        mkt = maskt[n:n + 1, :]
        houtt = mkt * hn2t + (1.0 - mkt) * ht_list[n]
        ht_scr[n] = houtt
        hout = jax.lax.dot_general(houtt, i256f_ref[...], _DN00,
                                   preferred_element_type=jnp.float32,
                                   precision=jax.lax.Precision.HIGHEST)
        out_ref[0, :, n * _BH:(n + 1) * _BH] = hout


def _full(shape):
    nd = len(shape)
    return pl.BlockSpec(shape, lambda t, _nd=nd: (0,) * _nd)


def kernel(input, hidden, seq_len, Wq_in, bq_in, Wk_in, Wv_in, Wx, Wh, bg,
           Wq_c, Wk_c, Wv_c):
    seq, batch, ninp = input.shape
    rows = seq * batch
    tile = 256
    grid_pre = rows // tile

    xf = input.reshape(rows, ninp).astype(jnp.bfloat16)
    kx_flat, v0_flat = pl.pallas_call(
        _precompute_body,
        grid=(grid_pre,),
        in_specs=[
            pl.BlockSpec((tile, ninp), lambda i: (i, 0)),
            pl.BlockSpec((ninp, _DK), lambda i: (0, 0)),
            pl.BlockSpec((ninp, _DV), lambda i: (0, 0)),
        ],
        out_specs=[
            pl.BlockSpec((tile, _DK), lambda i: (i, 0)),
            pl.BlockSpec((tile, _DV), lambda i: (i, 0)),
        ],
        out_shape=[
            jax.ShapeDtypeStruct((rows, _DK), jnp.bfloat16),
            jax.ShapeDtypeStruct((rows, _DV), jnp.bfloat16),
        ],
    )(xf, Wk_in.astype(jnp.bfloat16), Wv_in.astype(jnp.bfloat16))

    kx_v = kx_flat.reshape(seq, batch, _DK)
    v0_v = v0_flat.reshape(seq, batch, _DV)
    h0t = hidden.reshape(batch, _NB, _BH).transpose(1, 2, 0)

    bqt = jnp.broadcast_to(bq_in[:, :, None], (_NB, _DK, batch))
    bgt = jnp.broadcast_to(bg[:, :, None], (_NB, 3 * _BH, batch))
    i64b = jnp.eye(batch, dtype=jnp.bfloat16)
    i256f = jnp.eye(_BH, dtype=jnp.float32)

    in_specs = [
        pl.BlockSpec((1, batch, _DK), lambda t: (t, 0, 0)),
        pl.BlockSpec((1, batch, _DV), lambda t: (t, 0, 0)),
        _full((_NB, _BH, batch)),
    ]
    in_specs += [_full((_NB, _BH, _DK)), _full((_NB, _BH, 3 * _BH)),
                 _full((_NB, _DV, 3 * _BH)), _full((_NB, _BH, _DKC)),
                 _full((_NB, _BH, _DKC)), _full((_NB, _BH, _BH))]
    in_specs += [_full((_NB, _DK, batch)), _full((_NB, 3 * _BH, batch))]
    in_specs += [_full((batch, batch)), _full((_BH, _BH))]

    out = pl.pallas_call(
        _step_body,
        grid=(seq,),
        in_specs=in_specs,
        out_specs=pl.BlockSpec((1, batch, _NB * _BH), lambda t: (t, 0, 0)),
        out_shape=jax.ShapeDtypeStruct((seq, batch, _NB * _BH), jnp.float32),
        scratch_shapes=[pltpu.VMEM((_NB, _BH, batch), jnp.float32)],
        compiler_params=pltpu.CompilerParams(
            dimension_semantics=("arbitrary",),
        ),
    )(kx_v, v0_v, h0t, Wq_in, Wh, Wx, Wq_c, Wk_c, Wv_c, bqt, bgt, i64b, i256f)

    return out
